# baseline (speedup 1.0000x reference)
import jax
import jax.numpy as jnp
from jax import lax
from jax.experimental import pallas as pl
from jax.experimental.pallas import tpu as pltpu
from jax.experimental.pallas import tpu_sc as plsc

FEATURE = 1000000
BATCH = 16384
N_FIELDS = 26
NUM_CORES = 2
NUM_SUBCORES = 16
NUM_WORKERS = NUM_CORES * NUM_SUBCORES
ROWS_PER_W = BATCH // NUM_WORKERS
IDX_PER_W = ROWS_PER_W * N_FIELDS
LANES = 16


def _sc_body(w_hbm, idx_hbm, out_hbm, idx_v, rows_v, out_v, sem1, sem2):
    wid = lax.axis_index("s") * NUM_CORES + lax.axis_index("c")
    base_i = wid * IDX_PER_W
    base_o = wid * ROWS_PER_W
    half = IDX_PER_W // 2

    pltpu.sync_copy(idx_hbm.at[pl.ds(base_i, IDX_PER_W)], idx_v)
    c1 = pltpu.async_copy(
        w_hbm.at[idx_v.at[pl.ds(0, half)]], rows_v.at[pl.ds(0, half)], sem1
    )
    c2 = pltpu.async_copy(
        w_hbm.at[idx_v.at[pl.ds(half, half)]], rows_v.at[pl.ds(half, half)], sem2
    )
    c1.wait()
    c2.wait()

    @pl.loop(0, ROWS_PER_W // LANES)
    def _chunk(i):
        b = i * LANES
        acc = rows_v[pl.ds(b, LANES)]
        for f in range(1, N_FIELDS):
            acc = acc + rows_v[pl.ds(f * ROWS_PER_W + b, LANES)]
        out_v[pl.ds(b, LANES)] = acc

    pltpu.sync_copy(out_v, out_hbm.at[pl.ds(base_o, ROWS_PER_W)])


@jax.jit
def kernel(inputs, w):
    idx_flat = (
        inputs.astype(jnp.int32)
        .reshape(NUM_WORKERS, ROWS_PER_W, N_FIELDS)
        .transpose(0, 2, 1)
        .reshape(-1)
    )
    mesh = plsc.VectorSubcoreMesh(core_axis_name="c", subcore_axis_name="s")
    out = pl.kernel(
        _sc_body,
        out_type=jax.ShapeDtypeStruct((BATCH,), jnp.float32),
        mesh=mesh,
        scratch_types=[
            pltpu.VMEM((IDX_PER_W,), jnp.int32),
            pltpu.VMEM((IDX_PER_W,), jnp.float32),
            pltpu.VMEM((ROWS_PER_W,), jnp.float32),
            pltpu.SemaphoreType.DMA,
            pltpu.SemaphoreType.DMA,
        ],
    )(w.reshape(-1), idx_flat)
    return out.reshape(BATCH, 1)
